# Initial kernel scaffold; baseline (speedup 1.0000x reference)
import jax
import jax.numpy as jnp
from jax.experimental import pallas as pl


def kernel(f, pos, edge_attr, ij_index, params, edge_index, graph_ids):
    raise NotImplementedError("write your pallas kernel here")


# speedup vs baseline: 1.0536x; 1.0536x over previous
import functools

import jax
import jax.numpy as jnp
from jax.experimental import pallas as pl
from jax.experimental.pallas import tpu as pltpu

N = 10000; E = 320000; FEAT = 128; CMID = 128; HEADS = 4; DKV = 32; DH = 8
RDIM = 32; EDIM = 4; NG = 64; NL = 2


def _heads_kernel(hrp_ref, hcp_ref, ij_ref, wc_ref, wr_ref, pred_ref, x_ref):
    hrp = hrp_ref[...]
    hcp = hcp_ref[...]
    ijx = ij_ref[...]
    wc = wc_ref[...]
    wr = wr_ref[...]
    ij1 = wc[0:21, 0:128]; ij1b = wc[21, 0:128]
    ij2 = wc[22:150, 0:128]; ij2b = wc[150, 0:128]
    l1 = wc[151:407, 0:256]; l1b = wc[407, 0:256]
    l2 = wc[408:664, 0:21]; l2b = wc[664, 0:21]
    ij = jax.nn.relu(ijx @ ij1 + ij1b) @ ij2 + ij2b
    xc = jnp.concatenate([hcp, ij], axis=-1)
    y = jax.nn.relu(xc @ l1 + l1b) @ l2 + l2b
    x = jax.nn.sigmoid(y)
    e1 = wr[0:21, 0:128]; e1b = wr[21, 0:128]
    e2 = wr[22:150, 0:128]; e2b = wr[150, 0:128]
    r1 = wr[151:407, 0:256]; r1b = wr[407, 0:256]
    r2 = wr[408:664, 0:1]; r2b = wr[664, 0:1]
    xe = jax.nn.relu(x @ e1 + e1b) @ e2 + e2b
    hx = jnp.concatenate([hrp, xe], axis=-1)
    pred = jax.nn.relu(hx @ r1 + r1b) @ r2 + r2b
    pred_ref[...] = pred
    x_ref[...] = x


def _pack_heads(pc, pr):
    def pad_to(a, rows, cols):
        a = jnp.asarray(a, jnp.float32)
        if a.ndim == 1:
            a = a[None, :]
        return jnp.pad(a, ((0, rows - a.shape[0]), (0, cols - a.shape[1])))
    wc = jnp.concatenate([
        pad_to(pc['ij1'], 21, 256), pad_to(pc['ij1b'], 1, 256),
        pad_to(pc['ij2'], 128, 256), pad_to(pc['ij2b'], 1, 256),
        pad_to(pc['l1'], 256, 256), pad_to(pc['l1b'], 1, 256),
        pad_to(pc['l2'], 256, 256), pad_to(pc['l2b'], 1, 256),
    ], axis=0)
    wr = jnp.concatenate([
        pad_to(pr['e1'], 21, 256), pad_to(pr['e1b'], 1, 256),
        pad_to(pr['e2'], 128, 256), pad_to(pr['e2b'], 1, 256),
        pad_to(pr['l1'], 256, 256), pad_to(pr['l1b'], 1, 256),
        pad_to(pr['l2'], 256, 256), pad_to(pr['l2b'], 1, 256),
    ], axis=0)
    return wc, wr


def _run_heads(hrp, hcp, ij_index, pc, pr):
    wc, wr = _pack_heads(pc, pr)
    pred, x = pl.pallas_call(
        _heads_kernel,
        out_shape=(
            jax.ShapeDtypeStruct((NG, 1), jnp.float32),
            jax.ShapeDtypeStruct((NG, 21), jnp.float32),
        ),
    )(hrp, hcp, ij_index, wc, wr)
    return pred, x


def kernel(f, pos, edge_attr, ij_index, params, edge_index, graph_ids):
    src = edge_index[0]; dst = edge_index[1]
    d = pos[dst] - pos[src]
    r = jnp.sqrt(jnp.sum(d * d, axis=-1, keepdims=True) + 1e-12)
    e = jnp.concatenate([edge_attr, r], axis=-1)
    h = f
    for p in params['layers']:
        re = jax.nn.relu(e @ p['R1'] + p['R1b'])
        rk = re @ p['R2k']; rv = re @ p['R2v']
        q = (h @ p['Wq'])[dst].reshape(E, HEADS, DH)
        k = ((h @ p['Wk'])[src] * rk).reshape(E, HEADS, DH)
        v = ((h @ p['Wv'])[src] * rv).reshape(E, HEADS, DH)
        logits = jnp.sum(q * k, axis=-1) / jnp.sqrt(float(DH))
        ex = jnp.exp(logits)
        den = jax.ops.segment_sum(ex, dst, num_segments=N)
        num = jax.ops.segment_sum(ex[..., None] * v, dst, num_segments=N)
        msg = (num / (den[..., None] + 1e-9)).reshape(N, DKV)
        h2 = h + msg @ p['Wo']
        mu = jnp.mean(h2, axis=-1, keepdims=True)
        var = jnp.var(h2, axis=-1, keepdims=True)
        h = jax.nn.relu((h2 - mu) / jnp.sqrt(var + 1e-5) * p['gamma'] + p['beta'])
    def conv(hh, p):
        re = jax.nn.relu(e @ p['R1'] + p['R1b'])
        rw = re @ p['R2']
        msg = (hh @ p['W'])[src] * rw
        agg = jax.ops.segment_sum(msg, dst, num_segments=N)
        return agg + hh @ p['Wself'] + p['bself']
    hr = conv(h, params['reg'])
    hc = conv(h, params['cls'])
    cnt = jnp.maximum(jax.ops.segment_sum(jnp.ones((N,), jnp.float32), graph_ids, num_segments=NG), 1.0)
    hrp = jax.ops.segment_sum(hr, graph_ids, num_segments=NG) / cnt[:, None]
    hcp = jax.ops.segment_sum(hc, graph_ids, num_segments=NG) / cnt[:, None]
    pred, x = _run_heads(hrp, hcp, ij_index, params['mlpc'], params['mlpr'])
    return (pred, x)


# speedup vs baseline: 6.2776x; 5.9580x over previous
import functools

import jax
import jax.numpy as jnp
from jax import lax
from jax.experimental import pallas as pl
from jax.experimental.pallas import tpu as pltpu
from jax.experimental.pallas import tpu_sc as plsc

N = 10000; E = 320000; FEAT = 128; CMID = 128; HEADS = 4; DKV = 32; DH = 8
RDIM = 32; EDIM = 4; NG = 64; NL = 2

_NC, _NS = 2, 16
_NW = _NC * _NS
_PERW = E // _NW
_GCH = 80
_NCHUNK = _PERW // _GCH
_NPAD = 10240
_HR = 5120
_TR = 6400


def _dot(a, b):
    return jnp.dot(a, b, precision=lax.Precision.HIGHEST)


_mesh = plsc.VectorSubcoreMesh(core_axis_name="c", subcore_axis_name="s")


def _geo_body(src_hbm, dst_hbm, pos_hbm, d_hbm, idx_s, idx_d, ps, pd, sem):
    wid = lax.axis_index("s") * _NC + lax.axis_index("c")
    base0 = wid * _PERW

    @pl.loop(0, _NCHUNK)
    def _chunk(ci):
        base = base0 + ci * _GCH
        pltpu.sync_copy(src_hbm.at[pl.ds(base, _GCH)], idx_s)
        pltpu.sync_copy(dst_hbm.at[pl.ds(base, _GCH)], idx_d)
        pltpu.async_copy(pos_hbm.at[idx_s], ps, sem).wait()
        pltpu.async_copy(pos_hbm.at[idx_d], pd, sem).wait()
        for r in range(_GCH):
            pd[r, :] = pd[r, :] - ps[r, :]
        pltpu.sync_copy(pd, d_hbm.at[pl.ds(base, _GCH), :])


@jax.jit
def _geo(src, dst, pos16):
    return pl.kernel(
        _geo_body,
        out_type=jax.ShapeDtypeStruct((E, 16), jnp.float32),
        mesh=_mesh,
        scratch_types=[
            pltpu.VMEM((_GCH,), jnp.int32),
            pltpu.VMEM((_GCH,), jnp.int32),
            pltpu.VMEM((_GCH, 16), jnp.float32),
            pltpu.VMEM((_GCH, 16), jnp.float32),
            pltpu.SemaphoreType.DMA,
        ],
        compiler_params=pltpu.CompilerParams(use_tc_tiling_on_sc=False),
    )(src, dst, pos16)


def _attn_body(lo, src_hbm, dst_hbm, qkv_hbm, rkv_hbm, u_hbm,
               idx_s, idx_d, qrow, srow, rk, w, zb, table, sem):
    c = lax.axis_index("c")
    s = lax.axis_index("s")
    z16 = jnp.zeros((16,), jnp.float32)
    for i in range(_GCH):
        for j in range(8):
            zb[i, pl.ds(16 * j, 16)] = z16
            w[i, pl.ds(16 * j, 16)] = z16
    for k in range(5):
        pltpu.sync_copy(zb, table.at[pl.ds(s * 400 + k * 80, 80), :])
    plsc.subcore_barrier()
    base0 = s * (E // _NS)
    cb = jnp.zeros((16,), jnp.int32) + c * _HR
    jk = jnp.full((16,), _HR, jnp.int32)
    inv = 0.3535533905932738

    @pl.loop(0, (E // _NS) // _GCH)
    def _chunk(ci):
        base = base0 + ci * _GCH
        pltpu.sync_copy(src_hbm.at[pl.ds(base, _GCH)], idx_s)
        pltpu.sync_copy(dst_hbm.at[pl.ds(base, _GCH)], idx_d)
        pltpu.async_copy(qkv_hbm.at[idx_d], qrow, sem).wait()
        pltpu.async_copy(qkv_hbm.at[idx_s], srow, sem).wait()
        pltpu.sync_copy(rkv_hbm.at[pl.ds(base, _GCH), :], rk)
        for g5 in range(_GCH // 16):
            ld = idx_d[pl.ds(16 * g5, 16)]
            loc = ld - cb
            m = (loc >= 0) & (loc < _HR)
            idx_d[pl.ds(16 * g5, 16)] = jnp.where(m, loc, jk)
        for g in range(_GCH // 16):
            rows = lax.iota(jnp.int32, 16) + (16 * g)
            exs = []
            for h in range(HEADS):
                acc = None
                for d8 in range(DH):
                    d = h * DH + d8
                    qv = plsc.load_gather(
                        qrow, [rows, jnp.full((16,), d, jnp.int32)])
                    kv = plsc.load_gather(
                        srow, [rows, jnp.full((16,), 32 + d, jnp.int32)])
                    rkx = plsc.load_gather(
                        rk, [rows, jnp.full((16,), lo + d, jnp.int32)])
                    t = qv * kv * rkx
                    acc = t if acc is None else acc + t
                ex = jnp.exp(acc * inv)
                plsc.store_scatter(
                    w, [rows, jnp.full((16,), 32 + h, jnp.int32)], ex)
                exs.append(ex)
            for d in range(DKV):
                vv = plsc.load_gather(
                    srow, [rows, jnp.full((16,), 64 + d, jnp.int32)])
                rvv = plsc.load_gather(
                    rk, [rows, jnp.full((16,), lo + 32 + d, jnp.int32)])
                plsc.store_scatter(
                    w, [rows, jnp.full((16,), d, jnp.int32)],
                    exs[d // DH] * vv * rvv)
        pltpu.sync_copy(w, table.at[idx_d], add=True)

    plsc.subcore_barrier()
    pltpu.sync_copy(table.at[pl.ds(s * 320, 320), :],
                    u_hbm.at[pl.ds(c * _HR + s * 320, 320), :])


def _attn(lo, src, dst, qkvtab, rkvcat):
    return pl.kernel(
        functools.partial(_attn_body, lo),
        out_type=jax.ShapeDtypeStruct((_NPAD, 128), jnp.float32),
        mesh=_mesh,
        scratch_types=[
            pltpu.VMEM((_GCH,), jnp.int32),
            pltpu.VMEM((_GCH,), jnp.int32),
            pltpu.VMEM((_GCH, 128), jnp.float32),
            pltpu.VMEM((_GCH, 128), jnp.float32),
            pltpu.VMEM((_GCH, 128), jnp.float32),
            pltpu.VMEM((_GCH, 128), jnp.float32),
            pltpu.VMEM((_GCH, 128), jnp.float32),
            pltpu.VMEM_SHARED((_TR, 128), jnp.float32),
            pltpu.SemaphoreType.DMA,
        ],
        compiler_params=pltpu.CompilerParams(needs_layout_passes=False),
    )(src, dst, qkvtab, rkvcat)


def _conv_body(half, src_hbm, dst_hbm, gtab_hbm, rw_hbm,
               agg_hbm, idx_s, idx_d, g, rw, zb, table, sem):
    c = lax.axis_index("c")
    s = lax.axis_index("s")
    wid = s * _NC + c
    z16 = jnp.zeros((16,), jnp.float32)
    for i in range(_GCH):
        for j in range(8):
            zb[i, pl.ds(16 * j, 16)] = z16
    for k in range(5):
        pltpu.sync_copy(zb, table.at[pl.ds(s * 400 + k * 80, 80), :])
    plsc.subcore_barrier()
    base0 = wid * _PERW
    jk = jnp.full((16,), _HR, jnp.int32)

    @pl.loop(0, _NCHUNK)
    def _chunk(ci):
        base = base0 + ci * _GCH
        pltpu.sync_copy(src_hbm.at[pl.ds(base, _GCH)], idx_s)
        pltpu.sync_copy(dst_hbm.at[pl.ds(base, _GCH)], idx_d)
        pltpu.async_copy(gtab_hbm.at[idx_s], g, sem).wait()
        pltpu.sync_copy(rw_hbm.at[pl.ds(base, _GCH), :], rw)

        for g5 in range(_GCH // 16):
            ld = idx_d[pl.ds(16 * g5, 16)]
            loc = ld - (half * _HR)
            m = (loc >= 0) & (loc < _HR)
            idx_d[pl.ds(16 * g5, 16)] = jnp.where(m, loc, jk)

        for i in range(_GCH):
            ri = jnp.full((16,), i, jnp.int32)
            for j in range(8):
                col = lax.iota(jnp.int32, 16) + (16 * j)
                gv = plsc.load_gather(g, [ri, col])
                rv = plsc.load_gather(rw, [ri, col])
                plsc.store_scatter(g, [ri, col], gv * rv)

        pltpu.sync_copy(g, table.at[idx_d], add=True)

    plsc.subcore_barrier()
    pltpu.sync_copy(table.at[pl.ds(s * 320, 320), :],
                    agg_hbm.at[c, pl.ds(s * 320, 320), :])


def _conv(half, src, dst, gtab, rw):
    return pl.kernel(
        functools.partial(_conv_body, half),
        out_type=jax.ShapeDtypeStruct((2, _HR, 128), jnp.float32),
        mesh=_mesh,
        scratch_types=[
            pltpu.VMEM((_GCH,), jnp.int32),
            pltpu.VMEM((_GCH,), jnp.int32),
            pltpu.VMEM((_GCH, 128), jnp.float32),
            pltpu.VMEM((_GCH, 128), jnp.float32),
            pltpu.VMEM((_GCH, 128), jnp.float32),
            pltpu.VMEM_SHARED((_TR, 128), jnp.float32),
            pltpu.SemaphoreType.DMA,
        ],
        compiler_params=pltpu.CompilerParams(needs_layout_passes=False),
    )(src, dst, gtab, rw)


_BE = 2000


def _edgemid_body(d_ref, attr_ref, r1a_ref, r1r_ref, r1b_ref,
                  r2kv0_ref, r2kv1_ref, r2r_ref, r2c_ref,
                  rkv_ref, rwr_ref, rwc_ref):
    dd = d_ref[...]
    ssq = jnp.sum(dd * dd, axis=1, keepdims=True)
    r = jnp.sqrt(ssq + 1e-12)
    attr = attr_ref[...]

    def re(t):
        return jax.nn.relu(_dot(attr, r1a_ref[t]) + r * r1r_ref[t, 0:1, :]
                           + r1b_ref[t, 0:1, :])

    rkv_ref[...] = jnp.concatenate(
        [_dot(re(0), r2kv0_ref[...]), _dot(re(1), r2kv1_ref[...])], axis=1)
    rwr_ref[...] = _dot(re(2), r2r_ref[...])
    rwc_ref[...] = _dot(re(3), r2c_ref[...])


def _edgemid(d16, attr16, r1a, r1r, r1b, r2kv0, r2kv1, r2r, r2c):
    nblk = E // _BE
    wspec = lambda shp: pl.BlockSpec(shp, lambda i: tuple(0 for _ in shp))
    espec = lambda w: pl.BlockSpec((_BE, w), lambda i: (i, 0))
    return pl.pallas_call(
        _edgemid_body,
        grid=(nblk,),
        in_specs=[espec(16), espec(16), wspec((4, 16, 32)), wspec((4, 8, 32)),
                  wspec((4, 8, 32)), wspec((32, 64)), wspec((32, 64)),
                  wspec((32, 128)), wspec((32, 128))],
        out_specs=[espec(128), espec(128), espec(128)],
        out_shape=[jax.ShapeDtypeStruct((E, 128), jnp.float32),
                   jax.ShapeDtypeStruct((E, 128), jnp.float32),
                   jax.ShapeDtypeStruct((E, 128), jnp.float32)],
    )(d16, attr16, r1a, r1r, r1b, r2kv0, r2kv1, r2r, r2c)


def _proj_body(x_ref, w_ref, o_ref):
    o_ref[...] = _dot(x_ref[...], w_ref[...])


def _proj(x, w):
    return pl.pallas_call(
        _proj_body,
        out_shape=jax.ShapeDtypeStruct((x.shape[0], w.shape[1]), jnp.float32),
        compiler_params=pltpu.CompilerParams(vmem_limit_bytes=100 * 2**20),
    )(x, w)


def _mid_body(h_ref, u_ref, wo_ref, g_ref, b_ref, wp_ref, h_out, p_out):
    u = u_ref[0:N, :]
    W = u[:, 0:32]
    Dh = u[:, 32:36]
    hsel = (lax.broadcasted_iota(jnp.int32, (4, 32), 1) // 8
            == lax.broadcasted_iota(jnp.int32, (4, 32), 0)).astype(jnp.float32)
    den = _dot(Dh, hsel) + 1e-9
    msg = W / den
    h2 = h_ref[...] + _dot(msg, wo_ref[...])
    mu = jnp.mean(h2, axis=1, keepdims=True)
    xc = h2 - mu
    var = jnp.mean(xc * xc, axis=1, keepdims=True)
    hn = jax.nn.relu(xc * lax.rsqrt(var + 1e-5) * g_ref[...] + b_ref[...])
    h_out[...] = hn
    p_out[...] = _dot(hn, wp_ref[...])


def _mid(h, u, wo, gamma, beta, wp):
    return pl.pallas_call(
        _mid_body,
        out_shape=[jax.ShapeDtypeStruct((N, CMID), jnp.float32),
                   jax.ShapeDtypeStruct((N, wp.shape[1]), jnp.float32)],
        compiler_params=pltpu.CompilerParams(vmem_limit_bytes=100 * 2**20),
    )(h, u, wo, gamma, beta, wp)


def _final_body(agg_ref, h_ref, ws_ref, bs_ref, gid_ref, ij_ref, wc_ref,
                wr_ref, pred_ref, x_ref):
    h2 = h_ref[...]
    self2 = _dot(h2, ws_ref[...]) + bs_ref[...]
    hr = agg_ref[0, 0:N, :] + self2[:, 0:128]
    hc = agg_ref[1, 0:N, :] + self2[:, 128:256]
    gid = gid_ref[...]
    io = lax.broadcasted_iota(jnp.int32, (NG, 1), 0).astype(jnp.float32)
    onehot = (gid == io).astype(jnp.float32)
    cnt = jnp.maximum(jnp.sum(onehot, axis=1, keepdims=True), 1.0)
    hrp = _dot(onehot, hr) / cnt
    hcp = _dot(onehot, hc) / cnt
    ijx = ij_ref[...]
    wc = wc_ref[...]
    wr = wr_ref[...]
    ij1 = wc[0:21, 0:128]; ij1b = wc[21, 0:128]
    ij2 = wc[22:150, 0:128]; ij2b = wc[150, 0:128]
    l1 = wc[151:407, 0:256]; l1b = wc[407, 0:256]
    l2 = wc[408:664, 0:21]; l2b = wc[664, 0:21]
    ij = _dot(jax.nn.relu(_dot(ijx, ij1) + ij1b), ij2) + ij2b
    xc_ = jnp.concatenate([hcp, ij], axis=-1)
    y = _dot(jax.nn.relu(_dot(xc_, l1) + l1b), l2) + l2b
    x = jax.nn.sigmoid(y)
    e1 = wr[0:21, 0:128]; e1b = wr[21, 0:128]
    e2 = wr[22:150, 0:128]; e2b = wr[150, 0:128]
    r1 = wr[151:407, 0:256]; r1b = wr[407, 0:256]
    r2 = wr[408:664, 0:1]; r2b = wr[664, 0:1]
    xe = _dot(jax.nn.relu(_dot(x, e1) + e1b), e2) + e2b
    hx = jnp.concatenate([hrp, xe], axis=-1)
    pred = _dot(jax.nn.relu(_dot(hx, r1) + r1b), r2) + r2b
    pred_ref[...] = pred
    x_ref[...] = x


def _pack_heads(pc, pr):
    def pad_to(a, rows, cols):
        a = jnp.asarray(a, jnp.float32)
        if a.ndim == 1:
            a = a[None, :]
        return jnp.pad(a, ((0, rows - a.shape[0]), (0, cols - a.shape[1])))
    wc = jnp.concatenate([
        pad_to(pc['ij1'], 21, 256), pad_to(pc['ij1b'], 1, 256),
        pad_to(pc['ij2'], 128, 256), pad_to(pc['ij2b'], 1, 256),
        pad_to(pc['l1'], 256, 256), pad_to(pc['l1b'], 1, 256),
        pad_to(pc['l2'], 256, 256), pad_to(pc['l2b'], 1, 256),
    ], axis=0)
    wr = jnp.concatenate([
        pad_to(pr['e1'], 21, 256), pad_to(pr['e1b'], 1, 256),
        pad_to(pr['e2'], 128, 256), pad_to(pr['e2b'], 1, 256),
        pad_to(pr['l1'], 256, 256), pad_to(pr['l1b'], 1, 256),
        pad_to(pr['l2'], 256, 256), pad_to(pr['l2b'], 1, 256),
    ], axis=0)
    return wc, wr


def _final(agg, h2, ws, bs, gidf, ij_index, wc, wr):
    return pl.pallas_call(
        _final_body,
        out_shape=[jax.ShapeDtypeStruct((NG, 1), jnp.float32),
                   jax.ShapeDtypeStruct((NG, 21), jnp.float32)],
        compiler_params=pltpu.CompilerParams(vmem_limit_bytes=100 * 2**20),
    )(agg, h2, ws, bs, gidf, ij_index, wc, wr)


def kernel(f, pos, edge_attr, ij_index, params, edge_index, graph_ids):
    src = edge_index[0]
    dst = edge_index[1]
    pos16 = jnp.pad(pos, ((0, 0), (0, 13)))
    d16 = _geo(src, dst, pos16)

    attr16 = jnp.pad(edge_attr, ((0, 0), (0, 12)))
    L = params['layers']
    pr_, pc_ = params['reg'], params['cls']

    def pack_r1(p):
        r1 = jnp.asarray(p['R1'], jnp.float32)
        a = jnp.pad(r1[:4], ((0, 12), (0, 0)))
        rrow = jnp.pad(r1[4:5], ((0, 7), (0, 0)))
        b = jnp.pad(p['R1b'][None, :], ((0, 7), (0, 0)))
        return a, rrow, b

    packs = [pack_r1(L[0]), pack_r1(L[1]), pack_r1(pr_), pack_r1(pc_)]
    r1a = jnp.stack([p[0] for p in packs])
    r1r = jnp.stack([p[1] for p in packs])
    r1b = jnp.stack([p[2] for p in packs])
    r2kv0 = jnp.concatenate([L[0]['R2k'], L[0]['R2v']], axis=1)
    r2kv1 = jnp.concatenate([L[1]['R2k'], L[1]['R2v']], axis=1)
    rkvcat, rwr, rwc = _edgemid(d16, attr16, r1a, r1r, r1b,
                                r2kv0, r2kv1, pr_['R2'], pc_['R2'])

    def wqkv(p):
        w = jnp.concatenate([p['Wq'], p['Wk'], p['Wv']], axis=1)
        return jnp.pad(w, ((0, 0), (0, 32)))

    h = f
    qkvtab = _proj(h, wqkv(L[0]))
    for li, p in enumerate(L):
        u = _attn(64 * li, src, dst, qkvtab, rkvcat)
        if li + 1 < NL:
            wp = wqkv(L[li + 1])
        else:
            wp = jnp.concatenate([pr_['W'], pc_['W']], axis=1)
        gamma = p['gamma'][None, :]
        beta = p['beta'][None, :]
        h, qkvtab = _mid(h, u, p['Wo'], gamma, beta, wp)

    greg = qkvtab[:, 0:128]
    gcls = qkvtab[:, 128:256]
    ar0 = _conv(0, src, dst, greg, rwr)
    ar1 = _conv(1, src, dst, greg, rwr)
    ac0 = _conv(0, src, dst, gcls, rwc)
    ac1 = _conv(1, src, dst, gcls, rwc)
    aggr = jnp.concatenate([ar0[0] + ar0[1], ar1[0] + ar1[1]], axis=0)
    aggc = jnp.concatenate([ac0[0] + ac0[1], ac1[0] + ac1[1]], axis=0)
    agg = jnp.stack([aggr, aggc])

    ws = jnp.concatenate([pr_['Wself'], pc_['Wself']], axis=1)
    bs = jnp.concatenate([pr_['bself'], pc_['bself']])[None, :]
    gidf = graph_ids.astype(jnp.float32)[None, :]
    wc, wr = _pack_heads(params['mlpc'], params['mlpr'])
    pred, x = _final(agg, h, ws, bs, gidf, ij_index, wc, wr)
    return (pred, x)
